# baseline (speedup 1.0000x reference)
import functools

import jax
import jax.numpy as jnp
import numpy as np
from jax import lax
from jax.experimental import pallas as pl
from jax.experimental.pallas import tpu as pltpu
from jax.experimental.pallas import tpu_sc as plsc

N, E, D = 10000, 320000, 128
NC, NS = 2, 16
NW = NC * NS
EPW = E // NW
CH = 40
NCHUNK = EPW // CH
NPAD = 10112
RPT = NPAD // NS
LANES = 16
DH = D // 2


def _edge_mlp_body(a_ref, w0_ref, b0_ref, w_ref, b_ref, e_ref):
    a = a_ref[...]
    t = jnp.maximum(a * w0_ref[...] + b0_ref[...], 0.0)
    r = (jnp.dot(t.astype(jnp.bfloat16), w_ref[...],
                 preferred_element_type=jnp.float32)
         + b_ref[...]).astype(jnp.bfloat16).astype(jnp.float32)
    r32 = jax.lax.bitcast_convert_type(r, jnp.uint32)
    packed = (r32[:, :DH] >> 16) | (r32[:, DH:] & jnp.uint32(0xFFFF0000))
    e_ref[...] = jax.lax.bitcast_convert_type(packed, jnp.int32)


def _edge_mlp(edge_attr, w0, b0, wcat, bcat):
    BE = 8000
    grid = E // BE
    return pl.pallas_call(
        _edge_mlp_body,
        grid=(grid,),
        in_specs=[
            pl.BlockSpec((BE, 1), lambda i: (i, 0)),
            pl.BlockSpec((1, D), lambda i: (0, 0)),
            pl.BlockSpec((1, D), lambda i: (0, 0)),
            pl.BlockSpec((D, D), lambda i: (0, 0)),
            pl.BlockSpec((1, D), lambda i: (0, 0)),
        ],
        out_specs=pl.BlockSpec((BE, DH), lambda i: (i, 0)),
        out_shape=jax.ShapeDtypeStruct((E, DH), jnp.int32),
    )(edge_attr, w0, b0.reshape(1, D), wcat, bcat.reshape(1, D))


def _node_mlp_body(x_ref, p_ref, w1_ref, b1_ref, w2_ref, b2_ref, o_ref):
    h = x_ref[...] + p_ref[0] + p_ref[1]
    h = jnp.maximum(
        jnp.dot(h, w1_ref[...], preferred_element_type=jnp.float32)
        + b1_ref[...], 0.0)
    o_ref[...] = jnp.maximum(
        jnp.dot(h, w2_ref[...], preferred_element_type=jnp.float32)
        + b2_ref[...], 0.0)


def _node_mlp(x, p, w1, b1, w2, b2):
    BN = 2000
    grid = N // BN
    return pl.pallas_call(
        _node_mlp_body,
        grid=(grid,),
        in_specs=[
            pl.BlockSpec((BN, D), lambda i: (i, 0)),
            pl.BlockSpec((NC, BN, D), lambda i: (0, i, 0)),
            pl.BlockSpec((D, D), lambda i: (0, 0)),
            pl.BlockSpec((1, D), lambda i: (0, 0)),
            pl.BlockSpec((D, D), lambda i: (0, 0)),
            pl.BlockSpec((1, D), lambda i: (0, 0)),
        ],
        out_specs=pl.BlockSpec((BN, D), lambda i: (i, 0)),
        out_shape=jax.ShapeDtypeStruct((N, D), jnp.float32),
    )(x, p, w1, b1.reshape(1, D), w2, b2.reshape(1, D))


def _sc_aggr_body(x_hbm, src_hbm, dst_hbm, e_hbm, zeros_hbm, out_hbm,
                  srcall_v, xg0, xg1, xg2, xg3, e0, e1, d0, d1, d2, d3,
                  aggr_sh,
                  sg0, sg1, sg2, sg3, se0, se1, sd0, sd1, sd2, sd3,
                  ss0, ss1, ss2, ss3, sidx):
    c = lax.axis_index("c")
    s = lax.axis_index("s")
    wid = s * NC + c
    xg = (xg0, xg1, xg2, xg3)
    ev = (e0, e1)
    dv = (d0, d1, d2, d3)
    sg = (sg0, sg1, sg2, sg3)
    se = (se0, se1)
    sd = (sd0, sd1, sd2, sd3)
    ss = (ss0, ss1, ss2, ss3)

    pltpu.async_copy(src_hbm.at[pl.ds(wid * EPW, EPW)], srcall_v, sidx)
    pltpu.sync_copy(zeros_hbm.at[pl.ds(s * RPT, RPT)],
                    aggr_sh.at[pl.ds(s * RPT, RPT)])
    pltpu.make_async_copy(src_hbm.at[pl.ds(0, EPW)], srcall_v, sidx).wait()
    plsc.subcore_barrier()

    def fire_gather(q, k):
        pltpu.async_copy(x_hbm.at[srcall_v.at[pl.ds(q * CH, CH)]],
                         xg[k], sg[k])

    def fire_e_dst(q, k):
        base = wid * EPW + q * CH
        pltpu.async_copy(e_hbm.at[pl.ds(base, CH)], ev[k % 2], se[k % 2])
        pltpu.async_copy(dst_hbm.at[pl.ds(base, CH)], dv[k], sd[k])

    def wait_scatter(k):
        pltpu.make_async_copy(xg[k], aggr_sh.at[dv[k]], ss[k]).wait()

    def substep(q, k, do_wait_scatter, do_fire):
        b = k % 2
        k2 = (k + 2) % 4
        if do_wait_scatter:
            wait_scatter(k2)
        if do_fire:
            fire_gather(q + 2, k2)
        pltpu.make_async_copy(x_hbm.at[pl.ds(0, CH)], xg[k], sg[k]).wait()
        pltpu.make_async_copy(e_hbm.at[pl.ds(0, CH)], ev[b], se[b]).wait()
        pltpu.make_async_copy(dst_hbm.at[pl.ds(0, CH)], dv[k], sd[k]).wait()

        xg_v = xg[k]
        e_v = ev[b]

        @plsc.parallel_loop(0, CH, step=1, unroll=8)
        def row(r):
            for g in range(D // (2 * LANES)):
                ei = e_v[r, pl.ds(g * LANES, LANES)]
                ea = jax.lax.bitcast_convert_type(ei << 16, jnp.float32)
                ec = jax.lax.bitcast_convert_type(
                    ei & jnp.int32(-65536), jnp.float32)
                sl0 = pl.ds(2 * g * LANES, LANES)
                sl1 = pl.ds((2 * g + 1) * LANES, LANES)
                xg_v[r, sl0] = jnp.maximum(xg_v[r, sl0] + ea, 0.0)
                xg_v[r, sl1] = jnp.maximum(xg_v[r, sl1] + ec, 0.0)

        pltpu.async_copy(xg[k], aggr_sh.at[dv[k]], ss[k], add=True)
        if do_fire:
            fire_e_dst(q + 2, k2)

    fire_gather(0, 0)
    fire_e_dst(0, 0)
    fire_gather(1, 1)
    fire_e_dst(1, 1)
    substep(0, 0, False, True)
    substep(1, 1, False, True)
    substep(2, 2, True, True)
    substep(3, 3, True, True)

    def outer(jj, carry):
        q0 = 4 * jj
        substep(q0 + 0, 0, True, True)
        substep(q0 + 1, 1, True, True)
        substep(q0 + 2, 2, True, True)
        substep(q0 + 3, 3, True, True)
        return carry

    lax.fori_loop(1, NCHUNK // 4, outer, 0)
    substep(NCHUNK - 2, 0, True, False)
    substep(NCHUNK - 1, 1, True, False)
    wait_scatter(0)
    wait_scatter(1)

    plsc.subcore_barrier()
    pltpu.sync_copy(aggr_sh.at[pl.ds(s * RPT, RPT)],
                    out_hbm.at[c, pl.ds(s * RPT, RPT)])


_sc_aggr = pl.kernel(
    _sc_aggr_body,
    out_type=jax.ShapeDtypeStruct((NC, NPAD, D), jnp.float32),
    mesh=plsc.VectorSubcoreMesh(
        core_axis_name="c", subcore_axis_name="s",
        num_cores=NC, num_subcores=NS),
    scratch_types=[
        pltpu.VMEM((EPW,), jnp.int32),
        pltpu.VMEM((CH, D), jnp.float32),
        pltpu.VMEM((CH, D), jnp.float32),
        pltpu.VMEM((CH, D), jnp.float32),
        pltpu.VMEM((CH, D), jnp.float32),
        pltpu.VMEM((CH, DH), jnp.int32),
        pltpu.VMEM((CH, DH), jnp.int32),
        pltpu.VMEM((CH,), jnp.int32),
        pltpu.VMEM((CH,), jnp.int32),
        pltpu.VMEM((CH,), jnp.int32),
        pltpu.VMEM((CH,), jnp.int32),
        pltpu.VMEM_SHARED((NPAD, D), jnp.float32),
        pltpu.SemaphoreType.DMA,
        pltpu.SemaphoreType.DMA,
        pltpu.SemaphoreType.DMA,
        pltpu.SemaphoreType.DMA,
        pltpu.SemaphoreType.DMA,
        pltpu.SemaphoreType.DMA,
        pltpu.SemaphoreType.DMA,
        pltpu.SemaphoreType.DMA,
        pltpu.SemaphoreType.DMA,
        pltpu.SemaphoreType.DMA,
        pltpu.SemaphoreType.DMA,
        pltpu.SemaphoreType.DMA,
        pltpu.SemaphoreType.DMA,
        pltpu.SemaphoreType.DMA,
        pltpu.SemaphoreType.DMA,
    ],
)


def kernel(x, edge_index, edge_attr, W_lin0, b_lin0, W_lin1, b_lin1,
           g0_W1, g0_b1, g0_W2, g0_b2, g1_W1, g1_b1, g1_W2, g1_b2):
    src = edge_index[0].astype(jnp.int32)
    dst = edge_index[1].astype(jnp.int32)
    zeros = jnp.zeros((NPAD, D), jnp.float32)

    permlo = np.empty((DH,), dtype=np.int32)
    permhi = np.empty((DH,), dtype=np.int32)
    for g in range(D // 32):
        for i in range(16):
            permlo[16 * g + i] = 32 * g + i
            permhi[16 * g + i] = 32 * g + 16 + i
    wcat = jnp.concatenate(
        [W_lin1[:, permlo], W_lin1[:, permhi]], axis=1).astype(jnp.bfloat16)
    bcat = jnp.concatenate([b_lin1[permlo], b_lin1[permhi]])
    e = _edge_mlp(edge_attr, W_lin0, b_lin0, wcat, bcat)

    p = _sc_aggr(x, src, dst, e, zeros)
    h = _node_mlp(x, p, g0_W1, g0_b1, g0_W2, g0_b2)

    p = _sc_aggr(h, src, dst, e, zeros)
    h = _node_mlp(h, p, g1_W1, g1_b1, g1_W2, g1_b2)
    return h


# speedup vs baseline: 1.0045x; 1.0045x over previous
import jax
import jax.numpy as jnp
import numpy as np
from jax import lax
from jax.experimental import pallas as pl
from jax.experimental.pallas import tpu as pltpu
from jax.experimental.pallas import tpu_sc as plsc

N, E, D = 10000, 320000, 128
NC, NS = 2, 16
NW = NC * NS
EPW = E // NW
CH = 40
NCHUNK = EPW // CH
NPAD = 10112
RPT = NPAD // NS
LANES = 16
DH = D // 2


def _edge_mlp_body(a_ref, w0_ref, b0_ref, w_ref, b_ref, e_ref):
    a = a_ref[...]
    t = jnp.maximum(a * w0_ref[...] + b0_ref[...], 0.0)
    r = (jnp.dot(t.astype(jnp.bfloat16), w_ref[...],
                 preferred_element_type=jnp.float32)
         + b_ref[...]).astype(jnp.bfloat16).astype(jnp.float32)
    r32 = jax.lax.bitcast_convert_type(r, jnp.uint32)
    packed = (r32[:, :DH] >> 16) | (r32[:, DH:] & jnp.uint32(0xFFFF0000))
    e_ref[...] = jax.lax.bitcast_convert_type(packed, jnp.int32)


def _edge_mlp(edge_attr, w0, b0, wcat, bcat):
    BE = 8000
    grid = E // BE
    return pl.pallas_call(
        _edge_mlp_body,
        grid=(grid,),
        in_specs=[
            pl.BlockSpec((BE, 1), lambda i: (i, 0)),
            pl.BlockSpec((1, D), lambda i: (0, 0)),
            pl.BlockSpec((1, D), lambda i: (0, 0)),
            pl.BlockSpec((D, D), lambda i: (0, 0)),
            pl.BlockSpec((1, D), lambda i: (0, 0)),
        ],
        out_specs=pl.BlockSpec((BE, DH), lambda i: (i, 0)),
        out_shape=jax.ShapeDtypeStruct((E, DH), jnp.int32),
    )(edge_attr, w0, b0.reshape(1, D), wcat, bcat.reshape(1, D))


def _node_mlp_body(x_ref, p_ref, w1_ref, b1_ref, w2_ref, b2_ref, o_ref):
    h = x_ref[...] + p_ref[0] + p_ref[1]
    h = jnp.maximum(
        jnp.dot(h, w1_ref[...], preferred_element_type=jnp.float32)
        + b1_ref[...], 0.0)
    o_ref[...] = jnp.maximum(
        jnp.dot(h, w2_ref[...], preferred_element_type=jnp.float32)
        + b2_ref[...], 0.0)


def _node_mlp(x, p, w1, b1, w2, b2):
    BN = 2000
    grid = N // BN
    return pl.pallas_call(
        _node_mlp_body,
        grid=(grid,),
        in_specs=[
            pl.BlockSpec((BN, D), lambda i: (i, 0)),
            pl.BlockSpec((NC, BN, D), lambda i: (0, i, 0)),
            pl.BlockSpec((D, D), lambda i: (0, 0)),
            pl.BlockSpec((1, D), lambda i: (0, 0)),
            pl.BlockSpec((D, D), lambda i: (0, 0)),
            pl.BlockSpec((1, D), lambda i: (0, 0)),
        ],
        out_specs=pl.BlockSpec((BN, D), lambda i: (i, 0)),
        out_shape=jax.ShapeDtypeStruct((N, D), jnp.float32),
    )(x, p, w1, b1.reshape(1, D), w2, b2.reshape(1, D))


def _sc_aggr_body(x_hbm, src_hbm, dst_hbm, e_hbm, zeros_hbm, out_hbm,
                  srcall_v, xg0, xg1, xg2, xg3, e0, e1, d0, d1, d2, d3,
                  aggr_sh,
                  sg0, sg1, sg2, sg3, se0, se1, sd0, sd1, sd2, sd3,
                  ss0, ss1, ss2, ss3, sidx):
    c = lax.axis_index("c")
    s = lax.axis_index("s")
    wid = s * NC + c
    xg = (xg0, xg1, xg2, xg3)
    ev = (e0, e1)
    dv = (d0, d1, d2, d3)
    sg = (sg0, sg1, sg2, sg3)
    se = (se0, se1)
    sd = (sd0, sd1, sd2, sd3)
    ss = (ss0, ss1, ss2, ss3)

    pltpu.async_copy(src_hbm.at[pl.ds(wid * EPW, EPW)], srcall_v, sidx)
    pltpu.sync_copy(zeros_hbm.at[pl.ds(s * RPT, RPT)],
                    aggr_sh.at[pl.ds(s * RPT, RPT)])
    pltpu.make_async_copy(src_hbm.at[pl.ds(0, EPW)], srcall_v, sidx).wait()
    plsc.subcore_barrier()

    def fire_gather(q, k):
        pltpu.async_copy(x_hbm.at[srcall_v.at[pl.ds(q * CH, CH)]],
                         xg[k], sg[k])

    def fire_e_dst(q, k):
        base = wid * EPW + q * CH
        pltpu.async_copy(e_hbm.at[pl.ds(base, CH)], ev[k % 2], se[k % 2])
        pltpu.async_copy(dst_hbm.at[pl.ds(base, CH)], dv[k], sd[k])

    def wait_scatter(k):
        pltpu.make_async_copy(xg[k], aggr_sh.at[dv[k]], ss[k]).wait()

    def substep(q, k, do_wait_scatter, do_fire):
        b = k % 2
        k2 = (k + 2) % 4
        if do_wait_scatter:
            wait_scatter(k2)
        if do_fire:
            fire_gather(q + 2, k2)
        pltpu.make_async_copy(x_hbm.at[pl.ds(0, CH)], xg[k], sg[k]).wait()
        pltpu.make_async_copy(e_hbm.at[pl.ds(0, CH)], ev[b], se[b]).wait()
        pltpu.make_async_copy(dst_hbm.at[pl.ds(0, CH)], dv[k], sd[k]).wait()

        xg_v = xg[k]
        e_v = ev[b]

        @plsc.parallel_loop(0, CH, step=1, unroll=4)
        def row(r):
            for g in range(D // (2 * LANES)):
                ei = e_v[r, pl.ds(g * LANES, LANES)]
                ea = jax.lax.bitcast_convert_type(ei << 16, jnp.float32)
                ec = jax.lax.bitcast_convert_type(
                    ei & jnp.int32(-65536), jnp.float32)
                sl0 = pl.ds(2 * g * LANES, LANES)
                sl1 = pl.ds((2 * g + 1) * LANES, LANES)
                xg_v[r, sl0] = jnp.maximum(xg_v[r, sl0] + ea, 0.0)
                xg_v[r, sl1] = jnp.maximum(xg_v[r, sl1] + ec, 0.0)

        pltpu.async_copy(xg[k], aggr_sh.at[dv[k]], ss[k], add=True)
        if do_fire:
            fire_e_dst(q + 2, k2)

    fire_gather(0, 0)
    fire_e_dst(0, 0)
    fire_gather(1, 1)
    fire_e_dst(1, 1)
    substep(0, 0, False, True)
    substep(1, 1, False, True)
    substep(2, 2, True, True)
    substep(3, 3, True, True)

    def outer(jj, carry):
        q0 = 4 * jj
        substep(q0 + 0, 0, True, True)
        substep(q0 + 1, 1, True, True)
        substep(q0 + 2, 2, True, True)
        substep(q0 + 3, 3, True, True)
        return carry

    lax.fori_loop(1, NCHUNK // 4, outer, 0)
    substep(NCHUNK - 2, 0, True, False)
    substep(NCHUNK - 1, 1, True, False)
    wait_scatter(0)
    wait_scatter(1)

    plsc.subcore_barrier()
    pltpu.sync_copy(aggr_sh.at[pl.ds(s * RPT, RPT)],
                    out_hbm.at[c, pl.ds(s * RPT, RPT)])


_sc_aggr = pl.kernel(
    _sc_aggr_body,
    out_type=jax.ShapeDtypeStruct((NC, NPAD, D), jnp.float32),
    mesh=plsc.VectorSubcoreMesh(
        core_axis_name="c", subcore_axis_name="s",
        num_cores=NC, num_subcores=NS),
    scratch_types=[
        pltpu.VMEM((EPW,), jnp.int32),
        pltpu.VMEM((CH, D), jnp.float32),
        pltpu.VMEM((CH, D), jnp.float32),
        pltpu.VMEM((CH, D), jnp.float32),
        pltpu.VMEM((CH, D), jnp.float32),
        pltpu.VMEM((CH, DH), jnp.int32),
        pltpu.VMEM((CH, DH), jnp.int32),
        pltpu.VMEM((CH,), jnp.int32),
        pltpu.VMEM((CH,), jnp.int32),
        pltpu.VMEM((CH,), jnp.int32),
        pltpu.VMEM((CH,), jnp.int32),
        pltpu.VMEM_SHARED((NPAD, D), jnp.float32),
        pltpu.SemaphoreType.DMA,
        pltpu.SemaphoreType.DMA,
        pltpu.SemaphoreType.DMA,
        pltpu.SemaphoreType.DMA,
        pltpu.SemaphoreType.DMA,
        pltpu.SemaphoreType.DMA,
        pltpu.SemaphoreType.DMA,
        pltpu.SemaphoreType.DMA,
        pltpu.SemaphoreType.DMA,
        pltpu.SemaphoreType.DMA,
        pltpu.SemaphoreType.DMA,
        pltpu.SemaphoreType.DMA,
        pltpu.SemaphoreType.DMA,
        pltpu.SemaphoreType.DMA,
        pltpu.SemaphoreType.DMA,
    ],
)


def kernel(x, edge_index, edge_attr, W_lin0, b_lin0, W_lin1, b_lin1,
           g0_W1, g0_b1, g0_W2, g0_b2, g1_W1, g1_b1, g1_W2, g1_b2):
    src = edge_index[0].astype(jnp.int32)
    dst = edge_index[1].astype(jnp.int32)
    zeros = jnp.zeros((NPAD, D), jnp.float32)

    permlo = np.empty((DH,), dtype=np.int32)
    permhi = np.empty((DH,), dtype=np.int32)
    for g in range(D // 32):
        for i in range(16):
            permlo[16 * g + i] = 32 * g + i
            permhi[16 * g + i] = 32 * g + 16 + i
    wcat = jnp.concatenate(
        [W_lin1[:, permlo], W_lin1[:, permhi]], axis=1).astype(jnp.bfloat16)
    bcat = jnp.concatenate([b_lin1[permlo], b_lin1[permhi]])
    e = _edge_mlp(edge_attr, W_lin0, b_lin0, wcat, bcat)

    p = _sc_aggr(x, src, dst, e, zeros)
    h = _node_mlp(x, p, g0_W1, g0_b1, g0_W2, g0_b2)

    p = _sc_aggr(h, src, dst, e, zeros)
    h = _node_mlp(h, p, g1_W1, g1_b1, g1_W2, g1_b2)
    return h
